# Initial kernel scaffold; baseline (speedup 1.0000x reference)
import jax
import jax.numpy as jnp
from jax.experimental import pallas as pl


def kernel(x, W1, b1, W2, b2):
    raise NotImplementedError("write your pallas kernel here")


# speedup vs baseline: 6.0882x; 6.0882x over previous
import functools

import jax
import jax.numpy as jnp
from jax.experimental import pallas as pl

N, D, H, E = 32768, 768, 128, 64
BN = 1024


def _fused_body(x_ref, w1_ref, b1_ref, w2_ref, b2_ref, out_ref):
    h = jnp.dot(x_ref[...], w1_ref[...],
                preferred_element_type=jnp.float32)
    h = jnp.maximum(h + b1_ref[...], 0.0)
    logits = jnp.dot(h, w2_ref[...],
                     preferred_element_type=jnp.float32) + b2_ref[...]
    col = jax.lax.broadcasted_iota(jnp.int32, logits.shape, 1)
    m1 = jnp.max(logits, axis=1, keepdims=True)
    i1 = jnp.min(jnp.where(logits == m1, col, E), axis=1, keepdims=True)
    rest = jnp.where(col == i1, -jnp.inf, logits)
    m2 = jnp.max(rest, axis=1, keepdims=True)
    i2 = jnp.min(jnp.where(rest == m2, col, E), axis=1, keepdims=True)
    m = jnp.maximum(m1, 0.0)
    e1 = jnp.exp(m1 - m)
    e2 = jnp.exp(m2 - m)
    zv = jnp.exp(-m)
    denom = e1 + e2 + (E - 2) * zv
    out_ref[...] = jnp.where(col == i1, e1,
                             jnp.where(col == i2, e2, zv)) / denom


@jax.jit
def kernel(x, W1, b1, W2, b2):
    grid = (N // BN,)
    return pl.pallas_call(
        _fused_body,
        grid=grid,
        in_specs=[
            pl.BlockSpec((BN, D), lambda i: (i, 0)),
            pl.BlockSpec((D, H), lambda i: (0, 0)),
            pl.BlockSpec((1, H), lambda i: (0, 0)),
            pl.BlockSpec((H, E), lambda i: (0, 0)),
            pl.BlockSpec((1, E), lambda i: (0, 0)),
        ],
        out_specs=pl.BlockSpec((BN, E), lambda i: (i, 0)),
        out_shape=jax.ShapeDtypeStruct((N, E), jnp.float32),
    )(x, W1, b1.reshape(1, H), W2, b2.reshape(1, E))
